# Initial kernel scaffold; baseline (speedup 1.0000x reference)
import functools

import jax
import jax.numpy as jnp
from jax import lax
from jax.experimental import pallas as pl
from jax.experimental.pallas import tpu as pltpu
from jax.experimental.pallas import tpu_sc as plsc

N = 320000
D = 128
S = 10000
S_PAD = 10240
EPS = 1e-5

NC = 2
NS = 16
NW = NC * NS
ROWS_W = N // NW
CHUNK = 400
SUB = 80
NSUB = CHUNK // SUB
NCHUNK = ROWS_W // CHUNK
TSLICE = S_PAD // NS

R = 512
NB = N // R
W = 128


def _sc_segment_sums(x_hbm, ids_hbm, zero_hbm, czero_hbm, ones_hbm,
                     sums_hbm, counts_hbm,
                     x_buf, idx_buf, ones_buf, table, ctable):
    c = lax.axis_index("c")
    s = lax.axis_index("s")
    wid = c * NS + s

    pltpu.sync_copy(zero_hbm, table.at[pl.ds(s * TSLICE, TSLICE), :])
    pltpu.sync_copy(czero_hbm, ctable.at[pl.ds(s * TSLICE, TSLICE), :])
    pltpu.sync_copy(ones_hbm, ones_buf)
    plsc.subcore_barrier()

    def chunk_body(k, carry):
        row0 = wid * ROWS_W + k * CHUNK
        idr0 = wid * (ROWS_W // SUB) + k * NSUB
        pltpu.sync_copy(x_hbm.at[pl.ds(row0, CHUNK), :], x_buf)
        pltpu.sync_copy(ids_hbm.at[pl.ds(idr0, NSUB), :], idx_buf)
        for j in range(NSUB):
            pltpu.sync_copy(x_buf.at[pl.ds(j * SUB, SUB), :],
                            table.at[idx_buf.at[j]], add=True)
            pltpu.sync_copy(ones_buf, ctable.at[idx_buf.at[j]], add=True)
        return carry

    lax.fori_loop(0, NCHUNK, chunk_body, 0)
    plsc.subcore_barrier()

    pltpu.sync_copy(table.at[pl.ds(s * TSLICE, TSLICE), :],
                    sums_hbm.at[c, pl.ds(s * TSLICE, TSLICE), :])
    pltpu.sync_copy(ctable.at[pl.ds(s * TSLICE, TSLICE), :],
                    counts_hbm.at[c, pl.ds(s * TSLICE, TSLICE), :])


_sc_call = functools.partial(
    pl.kernel,
    out_type=(jax.ShapeDtypeStruct((NC, S_PAD, D), jnp.float32),
              jax.ShapeDtypeStruct((NC, S_PAD, 16), jnp.float32)),
    mesh=plsc.VectorSubcoreMesh(core_axis_name="c", subcore_axis_name="s"),
    scratch_types=[
        pltpu.VMEM((CHUNK, D), jnp.float32),
        pltpu.VMEM((NSUB, SUB), jnp.int32),
        pltpu.VMEM((SUB, 16), jnp.float32),
        pltpu.VMEM_SHARED((S_PAD, D), jnp.float32),
        pltpu.VMEM_SHARED((S_PAD, 16), jnp.float32),
    ],
)(_sc_segment_sums)


def _mlp_kernel(sums_ref, counts_ref, w1_ref, b1_ref, w2_ref, b2_ref, h_ref):
    sums = sums_ref[0] + sums_ref[1]
    cnt = counts_ref[0, :, 0:1] + counts_ref[1, :, 0:1]
    mean = sums / jnp.maximum(cnt, 1.0)
    h = jnp.dot(mean, w1_ref[...], preferred_element_type=jnp.float32)
    h = jnp.maximum(h + b1_ref[...], 0.0)
    h = jnp.dot(h, w2_ref[...], preferred_element_type=jnp.float32)
    h_ref[...] = h + b2_ref[...]


def _pass2_kernel(ids_smem_ref, x_ref, idsf_ref, h_ref, gamma_ref, beta_ref,
                  out_ref):
    base = ids_smem_ref[0, 0]
    last = ids_smem_ref[0, R - 1]
    off0 = (base // 8) * 8
    nwin = (last - off0) // W + 1

    idsf = idsf_ref[...]
    iota = lax.broadcasted_iota(jnp.float32, (1, W), 1)

    def win_body(k, acc):
        off = off0 + k * W
        hw = h_ref[pl.ds(off, W), :]
        oh = (idsf == iota + off.astype(jnp.float32)).astype(jnp.bfloat16)
        return acc + jax.lax.dot_general(
            oh, hw.astype(jnp.bfloat16), (((1,), (0,)), ((), ())),
            preferred_element_type=jnp.float32)

    sel = lax.fori_loop(0, nwin, win_body, jnp.zeros((R, D), jnp.float32))

    o = x_ref[...] + sel
    mu = jnp.mean(o, axis=1, keepdims=True)
    d = o - mu
    var = jnp.mean(d * d, axis=1, keepdims=True)
    out_ref[...] = d * lax.rsqrt(var + EPS) * gamma_ref[...] + beta_ref[...]


def kernel(x, subnet_id, W1, b1, W2, b2, gamma, beta):
    ids = subnet_id.astype(jnp.int32)
    ids2d = ids.reshape(N // SUB, SUB)
    idsf = ids.astype(jnp.float32).reshape(N, 1)
    ids_blk = ids.reshape(NB, R)

    zero = jnp.zeros((TSLICE, D), jnp.float32)
    czero = jnp.zeros((TSLICE, 16), jnp.float32)
    ones = jnp.ones((SUB, 16), jnp.float32)

    sums, counts = _sc_call(x, ids2d, zero, czero, ones)

    h = pl.pallas_call(
        _mlp_kernel,
        out_shape=jax.ShapeDtypeStruct((S_PAD, D), jnp.float32),
    )(sums, counts, W1, b1.reshape(1, D), W2, b2.reshape(1, D))

    out = pl.pallas_call(
        _pass2_kernel,
        grid=(NB,),
        in_specs=[
            pl.BlockSpec((1, R), lambda i: (i, 0), memory_space=pltpu.SMEM),
            pl.BlockSpec((R, D), lambda i: (i, 0)),
            pl.BlockSpec((R, 1), lambda i: (i, 0)),
            pl.BlockSpec((S_PAD, D), lambda i: (0, 0)),
            pl.BlockSpec((1, D), lambda i: (0, 0)),
            pl.BlockSpec((1, D), lambda i: (0, 0)),
        ],
        out_specs=pl.BlockSpec((R, D), lambda i: (i, 0)),
        out_shape=jax.ShapeDtypeStruct((N, D), jnp.float32),
    )(ids_blk, x, idsf, h, gamma.reshape(1, D), beta.reshape(1, D))
    return out


# speedup vs baseline: 2.3598x; 2.3598x over previous
import functools

import jax
import jax.numpy as jnp
from jax import lax
from jax.experimental import pallas as pl
from jax.experimental.pallas import tpu as pltpu
from jax.experimental.pallas import tpu_sc as plsc

N = 320000
D = 128
S = 10000
S_PAD = 10240
EPS = 1e-5

NC = 2
NS = 16
NW = NC * NS
ROWS_W = N // NW
SUB = 80
NCHUNK = ROWS_W // SUB

R = 1280
NB = N // R
W = 128


def _sc_gather(h_hbm, ids_hbm, g_hbm, idx_buf, rows_buf):
    c = lax.axis_index("c")
    s = lax.axis_index("s")
    wid = c * NS + s

    def chunk_body(k, carry):
        pltpu.sync_copy(ids_hbm.at[wid, k], idx_buf)
        pltpu.sync_copy(h_hbm.at[idx_buf.at[0]], rows_buf)
        row0 = wid * ROWS_W + k * SUB
        pltpu.sync_copy(rows_buf, g_hbm.at[pl.ds(row0, SUB), :])
        return carry

    lax.fori_loop(0, NCHUNK, chunk_body, 0)


def _sc_gather_call(h, ids4d):
    fn = functools.partial(
        pl.kernel,
        out_type=jax.ShapeDtypeStruct((N, D), jnp.float32),
        mesh=plsc.VectorSubcoreMesh(core_axis_name="c", subcore_axis_name="s",
                                    num_cores=NC, num_subcores=NS),
        scratch_types=[
            pltpu.VMEM((1, SUB), jnp.int32),
            pltpu.VMEM((SUB, D), jnp.float32),
        ],
    )(_sc_gather)
    return fn(h, ids4d)


def _seg_mlp_kernel(ids_smem_ref, idsr_ref, x_ref, w1_ref, b1_ref, w2_ref,
                    b2_ref, h_ref, acc_ref, cacc_ref):
    i = pl.program_id(0)

    @pl.when(i == 0)
    def _init():
        acc_ref[...] = jnp.zeros((S_PAD, D), jnp.float32)
        cacc_ref[...] = jnp.zeros((S_PAD, D), jnp.float32)

    @pl.when(i < NB)
    def _accum():
        base = ids_smem_ref[0, 0, 0]
        last = ids_smem_ref[0, 0, R - 1]
        off0 = (base // 8) * 8
        nwin = (last - off0) // W + 1

        idsr = idsr_ref[0]
        iota = lax.broadcasted_iota(jnp.int32, (W, 1), 0).astype(jnp.float32)
        xbf = x_ref[...].astype(jnp.bfloat16)
        ones = jnp.ones((R, D), jnp.bfloat16)
        dims = (((1,), (0,)), ((), ()))

        def accum_win(off):
            oht = (iota + off.astype(jnp.float32) == idsr).astype(jnp.bfloat16)
            st = jax.lax.dot_general(oht, xbf, dims,
                                     preferred_element_type=jnp.float32)
            ct = jax.lax.dot_general(oht, ones, dims,
                                     preferred_element_type=jnp.float32)
            acc_ref[pl.ds(off, W), :] += st
            cacc_ref[pl.ds(off, W), :] += ct

        accum_win(off0)

        def win_body(k, carry):
            accum_win(off0 + k * W)
            return carry

        @pl.when(nwin > 1)
        def _extra():
            lax.fori_loop(1, nwin, win_body, 0)

    @pl.when(i == NB)
    def _mlp():
        mean = acc_ref[...] / jnp.maximum(cacc_ref[...], 1.0)
        h = jnp.dot(mean, w1_ref[...], preferred_element_type=jnp.float32)
        h = jnp.maximum(h + b1_ref[...], 0.0)
        h = jnp.dot(h, w2_ref[...], preferred_element_type=jnp.float32)
        h_ref[...] = h + b2_ref[...]


def _pass2_kernel(x_ref, g_ref, gamma_ref, beta_ref, out_ref):
    o = x_ref[...] + g_ref[...]
    dims = (((1,), (0,)), ((), ()))
    jd = jnp.full((D, D), 1.0 / D, dtype=jnp.bfloat16)
    mu = jax.lax.dot_general(o.astype(jnp.bfloat16), jd, dims,
                             preferred_element_type=jnp.float32)
    d = o - mu
    msq = jax.lax.dot_general((d * d).astype(jnp.bfloat16), jd, dims,
                              preferred_element_type=jnp.float32)
    rstd = lax.rsqrt(msq + EPS)
    out_ref[...] = d * rstd * gamma_ref[...] + beta_ref[...]


def kernel(x, subnet_id, W1, b1, W2, b2, gamma, beta):
    ids = subnet_id.astype(jnp.int32)
    ids4d = ids.reshape(NW, NCHUNK, 1, SUB)
    idsr = ids.astype(jnp.float32).reshape(NB, 1, R)
    ids_blk = ids.reshape(NB, 1, R)

    clamp = lambda i: (jnp.minimum(i, NB - 1), 0, 0)
    h = pl.pallas_call(
        _seg_mlp_kernel,
        grid=(NB + 1,),
        in_specs=[
            pl.BlockSpec((1, 1, R), clamp, memory_space=pltpu.SMEM),
            pl.BlockSpec((1, 1, R), clamp),
            pl.BlockSpec((R, D), lambda i: (jnp.minimum(i, NB - 1), 0)),
            pl.BlockSpec((D, D), lambda i: (0, 0)),
            pl.BlockSpec((1, D), lambda i: (0, 0)),
            pl.BlockSpec((D, D), lambda i: (0, 0)),
            pl.BlockSpec((1, D), lambda i: (0, 0)),
        ],
        out_specs=pl.BlockSpec((S_PAD, D), lambda i: (0, 0)),
        out_shape=jax.ShapeDtypeStruct((S_PAD, D), jnp.float32),
        scratch_shapes=[
            pltpu.VMEM((S_PAD, D), jnp.float32),
            pltpu.VMEM((S_PAD, D), jnp.float32),
        ],
    )(ids_blk, idsr, x, W1, b1.reshape(1, D), W2, b2.reshape(1, D))

    g = _sc_gather_call(h, ids4d)

    out = pl.pallas_call(
        _pass2_kernel,
        grid=(NB,),
        in_specs=[
            pl.BlockSpec((R, D), lambda i: (i, 0)),
            pl.BlockSpec((R, D), lambda i: (i, 0)),
            pl.BlockSpec((1, D), lambda i: (0, 0)),
            pl.BlockSpec((1, D), lambda i: (0, 0)),
        ],
        out_specs=pl.BlockSpec((R, D), lambda i: (i, 0)),
        out_shape=jax.ShapeDtypeStruct((N, D), jnp.float32),
    )(x, g, gamma.reshape(1, D), beta.reshape(1, D))
    return out


# speedup vs baseline: 3.0519x; 1.2933x over previous
import functools

import jax
import jax.numpy as jnp
from jax import lax
from jax.experimental import pallas as pl
from jax.experimental.pallas import tpu as pltpu
from jax.experimental.pallas import tpu_sc as plsc

N = 320000
D = 128
S = 10000
S_PAD = 10240
EPS = 1e-5

NC = 2
NS = 16
NW = NC * NS
ROWS_W = N // NW
SUB = 80
NSUB = 5
CHUNK = SUB * NSUB
NCHUNK = ROWS_W // CHUNK

R = 1280
NB = N // R
W = 128


def _sc_gather(h_hbm, ids_hbm, g_hbm, idx_buf, rows_buf, sem):
    c = lax.axis_index("c")
    s = lax.axis_index("s")
    wid = c * NS + s

    def chunk_body(k, carry):
        pltpu.sync_copy(ids_hbm.at[wid, k], idx_buf)
        copies = [
            pltpu.async_copy(h_hbm.at[idx_buf.at[j]],
                             rows_buf.at[pl.ds(j * SUB, SUB), :], sem)
            for j in range(NSUB)
        ]
        for cp in copies:
            cp.wait()
        row0 = wid * ROWS_W + k * CHUNK
        pltpu.sync_copy(rows_buf, g_hbm.at[pl.ds(row0, CHUNK), :])
        return carry

    lax.fori_loop(0, NCHUNK, chunk_body, 0)


def _sc_gather_call(h, ids4d):
    fn = functools.partial(
        pl.kernel,
        out_type=jax.ShapeDtypeStruct((N, D), jnp.float32),
        mesh=plsc.VectorSubcoreMesh(core_axis_name="c", subcore_axis_name="s",
                                    num_cores=NC, num_subcores=NS),
        scratch_types=[
            pltpu.VMEM((NSUB, SUB), jnp.int32),
            pltpu.VMEM((CHUNK, D), jnp.float32),
            pltpu.SemaphoreType.DMA,
        ],
    )(_sc_gather)
    return fn(h, ids4d)


def _seg_mlp_kernel(ids_smem_ref, idsr_ref, x_ref, w1_ref, b1_ref, w2_ref,
                    b2_ref, h_ref, acc_ref, cacc_ref):
    i = pl.program_id(0)

    @pl.when(i == 0)
    def _init():
        acc_ref[...] = jnp.zeros((S_PAD, D), jnp.float32)
        cacc_ref[...] = jnp.zeros((S_PAD, D), jnp.float32)

    @pl.when(i < NB)
    def _accum():
        base = ids_smem_ref[0, 0, 0]
        last = ids_smem_ref[0, 0, R - 1]
        off0 = (base // 8) * 8
        nwin = (last - off0) // W + 1

        idsr = idsr_ref[0]
        iota = lax.broadcasted_iota(jnp.int32, (W, 1), 0).astype(jnp.float32)
        xbf = x_ref[...].astype(jnp.bfloat16)
        ones = jnp.ones((R, D), jnp.bfloat16)
        dims = (((1,), (0,)), ((), ()))

        def accum_win(off):
            oht = (iota + off.astype(jnp.float32) == idsr).astype(jnp.bfloat16)
            st = jax.lax.dot_general(oht, xbf, dims,
                                     preferred_element_type=jnp.float32)
            ct = jax.lax.dot_general(oht, ones, dims,
                                     preferred_element_type=jnp.float32)
            acc_ref[pl.ds(off, W), :] += st
            cacc_ref[pl.ds(off, W), :] += ct

        accum_win(off0)

        def win_body(k, carry):
            accum_win(off0 + k * W)
            return carry

        @pl.when(nwin > 1)
        def _extra():
            lax.fori_loop(1, nwin, win_body, 0)

    @pl.when(i == NB)
    def _mlp():
        mean = acc_ref[...] / jnp.maximum(cacc_ref[...], 1.0)
        h = jnp.dot(mean, w1_ref[...], preferred_element_type=jnp.float32)
        h = jnp.maximum(h + b1_ref[...], 0.0)
        h = jnp.dot(h, w2_ref[...], preferred_element_type=jnp.float32)
        h_ref[...] = h + b2_ref[...]


def _pass2_kernel(x_ref, g_ref, gamma_ref, beta_ref, out_ref):
    o = x_ref[...] + g_ref[...]
    dims = (((1,), (0,)), ((), ()))
    jd = jnp.full((D, D), 1.0 / D, dtype=jnp.bfloat16)
    mu = jax.lax.dot_general(o.astype(jnp.bfloat16), jd, dims,
                             preferred_element_type=jnp.float32)
    d = o - mu
    msq = jax.lax.dot_general((d * d).astype(jnp.bfloat16), jd, dims,
                              preferred_element_type=jnp.float32)
    rstd = lax.rsqrt(msq + EPS)
    out_ref[...] = d * rstd * gamma_ref[...] + beta_ref[...]


def kernel(x, subnet_id, W1, b1, W2, b2, gamma, beta):
    ids = subnet_id.astype(jnp.int32)
    ids4d = ids.reshape(NW, NCHUNK, NSUB, SUB)
    idsr = ids.astype(jnp.float32).reshape(NB, 1, R)
    ids_blk = ids.reshape(NB, 1, R)

    clamp = lambda i: (jnp.minimum(i, NB - 1), 0, 0)
    h = pl.pallas_call(
        _seg_mlp_kernel,
        grid=(NB + 1,),
        in_specs=[
            pl.BlockSpec((1, 1, R), clamp, memory_space=pltpu.SMEM),
            pl.BlockSpec((1, 1, R), clamp),
            pl.BlockSpec((R, D), lambda i: (jnp.minimum(i, NB - 1), 0)),
            pl.BlockSpec((D, D), lambda i: (0, 0)),
            pl.BlockSpec((1, D), lambda i: (0, 0)),
            pl.BlockSpec((D, D), lambda i: (0, 0)),
            pl.BlockSpec((1, D), lambda i: (0, 0)),
        ],
        out_specs=pl.BlockSpec((S_PAD, D), lambda i: (0, 0)),
        out_shape=jax.ShapeDtypeStruct((S_PAD, D), jnp.float32),
        scratch_shapes=[
            pltpu.VMEM((S_PAD, D), jnp.float32),
            pltpu.VMEM((S_PAD, D), jnp.float32),
        ],
    )(ids_blk, idsr, x, W1, b1.reshape(1, D), W2, b2.reshape(1, D))

    g = _sc_gather_call(h, ids4d)

    out = pl.pallas_call(
        _pass2_kernel,
        grid=(NB,),
        in_specs=[
            pl.BlockSpec((R, D), lambda i: (i, 0)),
            pl.BlockSpec((R, D), lambda i: (i, 0)),
            pl.BlockSpec((1, D), lambda i: (0, 0)),
            pl.BlockSpec((1, D), lambda i: (0, 0)),
        ],
        out_specs=pl.BlockSpec((R, D), lambda i: (i, 0)),
        out_shape=jax.ShapeDtypeStruct((N, D), jnp.float32),
    )(x, g, gamma.reshape(1, D), beta.reshape(1, D))
    return out


# speedup vs baseline: 3.0565x; 1.0015x over previous
import functools

import jax
import jax.numpy as jnp
from jax import lax
from jax.experimental import pallas as pl
from jax.experimental.pallas import tpu as pltpu
from jax.experimental.pallas import tpu_sc as plsc

N = 320000
D = 128
S = 10000
S_PAD = 10240
EPS = 1e-5

NC = 2
NS = 16
NW = NC * NS
ROWS_W = N // NW
SUB = 80
NSUB = 5
CHUNK = SUB * NSUB
NCHUNK = ROWS_W // CHUNK

R = 1280
NB = N // R
W = 128


def _sc_gather(h_hbm, ids_hbm, g_hbm, idx_buf, rows_buf, sem):
    c = lax.axis_index("c")
    s = lax.axis_index("s")
    wid = c * NS + s

    def chunk_body(k, carry):
        pltpu.sync_copy(ids_hbm.at[wid, k], idx_buf)
        copies = [
            pltpu.async_copy(h_hbm.at[idx_buf.at[j]],
                             rows_buf.at[pl.ds(j * SUB, SUB), :], sem)
            for j in range(NSUB)
        ]
        for cp in copies:
            cp.wait()
        row0 = wid * ROWS_W + k * CHUNK
        pltpu.sync_copy(rows_buf, g_hbm.at[pl.ds(row0, CHUNK), :])
        return carry

    lax.fori_loop(0, NCHUNK, chunk_body, 0)


def _sc_gather_call(h, ids4d):
    fn = functools.partial(
        pl.kernel,
        out_type=jax.ShapeDtypeStruct((N, D), jnp.float32),
        mesh=plsc.VectorSubcoreMesh(core_axis_name="c", subcore_axis_name="s",
                                    num_cores=NC, num_subcores=NS),
        scratch_types=[
            pltpu.VMEM((NSUB, SUB), jnp.int32),
            pltpu.VMEM((CHUNK, D), jnp.float32),
            pltpu.SemaphoreType.DMA,
        ],
    )(_sc_gather)
    return fn(h, ids4d)


def _seg_mlp_kernel(ids_smem_ref, idsr_ref, x_ref, w1_ref, b1_ref, w2_ref,
                    b2_ref, h_ref, acc_ref, cacc_ref):
    i = pl.program_id(0)

    @pl.when(i == 0)
    def _init():
        acc_ref[...] = jnp.zeros((S_PAD, D), jnp.float32)
        cacc_ref[...] = jnp.zeros((S_PAD, D), jnp.float32)

    @pl.when(i < NB)
    def _accum():
        base = ids_smem_ref[0, 0, 0]
        last = ids_smem_ref[0, 0, R - 1]
        off0 = (base // 8) * 8
        nwin = (last - off0) // W + 1

        idsr = idsr_ref[0]
        iota = lax.broadcasted_iota(jnp.int32, (W, R), 0)
        xbf = x_ref[...].astype(jnp.bfloat16)
        ones = jnp.ones((R, D), jnp.bfloat16)
        dims = (((1,), (0,)), ((), ()))

        def accum_win(off):
            oht = (iota == idsr - off).astype(jnp.bfloat16)
            st = jax.lax.dot_general(oht, xbf, dims,
                                     preferred_element_type=jnp.float32)
            ct = jax.lax.dot_general(oht, ones, dims,
                                     preferred_element_type=jnp.float32)
            acc_ref[pl.ds(off, W), :] += st
            cacc_ref[pl.ds(off, W), :] += ct

        accum_win(off0)

        def win_body(k, carry):
            accum_win(off0 + k * W)
            return carry

        @pl.when(nwin > 1)
        def _extra():
            lax.fori_loop(1, nwin, win_body, 0)

    @pl.when(i == NB)
    def _mlp():
        mean = acc_ref[...] / jnp.maximum(cacc_ref[...], 1.0)
        h = jnp.dot(mean, w1_ref[...], preferred_element_type=jnp.float32)
        h = jnp.maximum(h + b1_ref[...], 0.0)
        h = jnp.dot(h, w2_ref[...], preferred_element_type=jnp.float32)
        h_ref[...] = h + b2_ref[...]


def _pass2_kernel(x_ref, g_ref, gamma_ref, beta_ref, out_ref):
    o = x_ref[...] + g_ref[...]
    dims = (((1,), (0,)), ((), ()))
    jd = jnp.full((D, D), 1.0 / D, dtype=jnp.bfloat16)
    mu = jax.lax.dot_general(o.astype(jnp.bfloat16), jd, dims,
                             preferred_element_type=jnp.float32)
    d = o - mu
    msq = jax.lax.dot_general((d * d).astype(jnp.bfloat16), jd, dims,
                              preferred_element_type=jnp.float32)
    rstd = lax.rsqrt(msq + EPS)
    out_ref[...] = d * rstd * gamma_ref[...] + beta_ref[...]


def kernel(x, subnet_id, W1, b1, W2, b2, gamma, beta):
    ids = subnet_id.astype(jnp.int32)
    ids4d = ids.reshape(NW, NCHUNK, NSUB, SUB)
    ids_blk = ids.reshape(NB, 1, R)

    clamp = lambda i: (jnp.minimum(i, NB - 1), 0, 0)
    h = pl.pallas_call(
        _seg_mlp_kernel,
        grid=(NB + 1,),
        in_specs=[
            pl.BlockSpec((1, 1, R), clamp, memory_space=pltpu.SMEM),
            pl.BlockSpec((1, 1, R), clamp),
            pl.BlockSpec((R, D), lambda i: (jnp.minimum(i, NB - 1), 0)),
            pl.BlockSpec((D, D), lambda i: (0, 0)),
            pl.BlockSpec((1, D), lambda i: (0, 0)),
            pl.BlockSpec((D, D), lambda i: (0, 0)),
            pl.BlockSpec((1, D), lambda i: (0, 0)),
        ],
        out_specs=pl.BlockSpec((S_PAD, D), lambda i: (0, 0)),
        out_shape=jax.ShapeDtypeStruct((S_PAD, D), jnp.float32),
        scratch_shapes=[
            pltpu.VMEM((S_PAD, D), jnp.float32),
            pltpu.VMEM((S_PAD, D), jnp.float32),
        ],
    )(ids_blk, ids_blk, x, W1, b1.reshape(1, D), W2, b2.reshape(1, D))

    g = _sc_gather_call(h, ids4d)

    out = pl.pallas_call(
        _pass2_kernel,
        grid=(NB,),
        in_specs=[
            pl.BlockSpec((R, D), lambda i: (i, 0)),
            pl.BlockSpec((R, D), lambda i: (i, 0)),
            pl.BlockSpec((1, D), lambda i: (0, 0)),
            pl.BlockSpec((1, D), lambda i: (0, 0)),
        ],
        out_specs=pl.BlockSpec((R, D), lambda i: (i, 0)),
        out_shape=jax.ShapeDtypeStruct((N, D), jnp.float32),
    )(x, g, gamma.reshape(1, D), beta.reshape(1, D))
    return out


# speedup vs baseline: 3.3069x; 1.0819x over previous
import functools

import jax
import jax.numpy as jnp
from jax import lax
from jax.experimental import pallas as pl
from jax.experimental.pallas import tpu as pltpu
from jax.experimental.pallas import tpu_sc as plsc

N = 320000
D = 128
S = 10000
S_PAD = 10240
EPS = 1e-5

NC = 2
NS = 16
NW = NC * NS
ROWS_W = N // NW
SUB = 80
NSUB = 5
CHUNK = SUB * NSUB
NCHUNK = ROWS_W // CHUNK

R = 1280
NB = N // R
W = 128


def _sc_gather(h_hbm, ids_hbm, g_hbm, idx0, idx1, rows0, rows1,
               gsem0, gsem1, wsem0, wsem1):
    c = lax.axis_index("c")
    s = lax.axis_index("s")
    wid = c * NS + s
    idx_bufs = (idx0, idx1)
    rows_bufs = (rows0, rows1)
    gsems = (gsem0, gsem1)
    wsems = (wsem0, wsem1)

    def fire(k, b):
        pltpu.sync_copy(ids_hbm.at[wid, k], idx_bufs[b])
        for j in range(NSUB):
            pltpu.async_copy(h_hbm.at[idx_bufs[b].at[j]],
                             rows_bufs[b].at[pl.ds(j * SUB, SUB), :], gsems[b])

    def drain_gathers(b):
        for j in range(NSUB):
            pltpu.make_async_copy(
                h_hbm.at[idx_bufs[b].at[j]],
                rows_bufs[b].at[pl.ds(j * SUB, SUB), :], gsems[b]).wait()

    def write(k, b):
        row0 = wid * ROWS_W + k * CHUNK
        pltpu.async_copy(rows_bufs[b], g_hbm.at[pl.ds(row0, CHUNK), :],
                         wsems[b])

    def drain_write(b):
        pltpu.make_async_copy(rows_bufs[b],
                              g_hbm.at[pl.ds(wid * ROWS_W, CHUNK), :],
                              wsems[b]).wait()

    fire(0, 0)

    def body(k, carry):
        for b in range(2):
            @pl.when(k % 2 == b)
            def _step():
                nxt = 1 - b

                @pl.when(k + 1 < NCHUNK)
                def _prefetch():
                    @pl.when(k >= 1)
                    def _dw():
                        drain_write(nxt)

                    fire(k + 1, nxt)

                drain_gathers(b)
                write(k, b)
        return carry

    lax.fori_loop(0, NCHUNK, body, 0)
    drain_write((NCHUNK - 1) % 2)
    drain_write(NCHUNK % 2)


def _sc_gather_call(h, ids4d):
    fn = functools.partial(
        pl.kernel,
        out_type=jax.ShapeDtypeStruct((N, D), jnp.float32),
        mesh=plsc.VectorSubcoreMesh(core_axis_name="c", subcore_axis_name="s",
                                    num_cores=NC, num_subcores=NS),
        scratch_types=[
            pltpu.VMEM((NSUB, SUB), jnp.int32),
            pltpu.VMEM((NSUB, SUB), jnp.int32),
            pltpu.VMEM((CHUNK, D), jnp.float32),
            pltpu.VMEM((CHUNK, D), jnp.float32),
            pltpu.SemaphoreType.DMA,
            pltpu.SemaphoreType.DMA,
            pltpu.SemaphoreType.DMA,
            pltpu.SemaphoreType.DMA,
        ],
    )(_sc_gather)
    return fn(h, ids4d)


def _seg_mlp_kernel(ids_smem_ref, idsr_ref, x_ref, w1_ref, b1_ref, w2_ref,
                    b2_ref, h_ref, acc_ref, cacc_ref):
    i = pl.program_id(0)

    @pl.when(i == 0)
    def _init():
        acc_ref[...] = jnp.zeros((S_PAD, D), jnp.float32)
        cacc_ref[...] = jnp.zeros((S_PAD, D), jnp.float32)

    @pl.when(i < NB)
    def _accum():
        base = ids_smem_ref[0, 0, 0]
        last = ids_smem_ref[0, 0, R - 1]
        off0 = (base // 8) * 8
        nwin = (last - off0) // W + 1

        idsr = idsr_ref[0]
        iota = lax.broadcasted_iota(jnp.int32, (W, R), 0)
        xbf = x_ref[...].astype(jnp.bfloat16)
        ones = jnp.ones((R, D), jnp.bfloat16)
        dims = (((1,), (0,)), ((), ()))

        def accum_win(off):
            oht = (iota == idsr - off).astype(jnp.bfloat16)
            st = jax.lax.dot_general(oht, xbf, dims,
                                     preferred_element_type=jnp.float32)
            ct = jax.lax.dot_general(oht, ones, dims,
                                     preferred_element_type=jnp.float32)
            acc_ref[pl.ds(off, W), :] += st
            cacc_ref[pl.ds(off, W), :] += ct

        accum_win(off0)

        def win_body(k, carry):
            accum_win(off0 + k * W)
            return carry

        @pl.when(nwin > 1)
        def _extra():
            lax.fori_loop(1, nwin, win_body, 0)

    @pl.when(i == NB)
    def _mlp():
        mean = acc_ref[...] / jnp.maximum(cacc_ref[...], 1.0)
        h = jnp.dot(mean, w1_ref[...], preferred_element_type=jnp.float32)
        h = jnp.maximum(h + b1_ref[...], 0.0)
        h = jnp.dot(h, w2_ref[...], preferred_element_type=jnp.float32)
        h_ref[...] = h + b2_ref[...]


def _pass2_kernel(x_ref, g_ref, gamma_ref, beta_ref, out_ref):
    o = x_ref[...] + g_ref[...]
    dims = (((1,), (0,)), ((), ()))
    jd = jnp.full((D, D), 1.0 / D, dtype=jnp.bfloat16)
    mu = jax.lax.dot_general(o.astype(jnp.bfloat16), jd, dims,
                             preferred_element_type=jnp.float32)
    d = o - mu
    msq = jax.lax.dot_general((d * d).astype(jnp.bfloat16), jd, dims,
                              preferred_element_type=jnp.float32)
    rstd = lax.rsqrt(msq + EPS)
    out_ref[...] = d * rstd * gamma_ref[...] + beta_ref[...]


def kernel(x, subnet_id, W1, b1, W2, b2, gamma, beta):
    ids = subnet_id.astype(jnp.int32)
    ids4d = ids.reshape(NW, NCHUNK, NSUB, SUB)
    ids_blk = ids.reshape(NB, 1, R)

    clamp = lambda i: (jnp.minimum(i, NB - 1), 0, 0)
    h = pl.pallas_call(
        _seg_mlp_kernel,
        grid=(NB + 1,),
        in_specs=[
            pl.BlockSpec((1, 1, R), clamp, memory_space=pltpu.SMEM),
            pl.BlockSpec((1, 1, R), clamp),
            pl.BlockSpec((R, D), lambda i: (jnp.minimum(i, NB - 1), 0)),
            pl.BlockSpec((D, D), lambda i: (0, 0)),
            pl.BlockSpec((1, D), lambda i: (0, 0)),
            pl.BlockSpec((D, D), lambda i: (0, 0)),
            pl.BlockSpec((1, D), lambda i: (0, 0)),
        ],
        out_specs=pl.BlockSpec((S_PAD, D), lambda i: (0, 0)),
        out_shape=jax.ShapeDtypeStruct((S_PAD, D), jnp.float32),
        scratch_shapes=[
            pltpu.VMEM((S_PAD, D), jnp.float32),
            pltpu.VMEM((S_PAD, D), jnp.float32),
        ],
    )(ids_blk, ids_blk, x, W1, b1.reshape(1, D), W2, b2.reshape(1, D))

    g = _sc_gather_call(h, ids4d)

    out = pl.pallas_call(
        _pass2_kernel,
        grid=(NB,),
        in_specs=[
            pl.BlockSpec((R, D), lambda i: (i, 0)),
            pl.BlockSpec((R, D), lambda i: (i, 0)),
            pl.BlockSpec((1, D), lambda i: (0, 0)),
            pl.BlockSpec((1, D), lambda i: (0, 0)),
        ],
        out_specs=pl.BlockSpec((R, D), lambda i: (i, 0)),
        out_shape=jax.ShapeDtypeStruct((N, D), jnp.float32),
    )(x, g, gamma.reshape(1, D), beta.reshape(1, D))
    return out


# speedup vs baseline: 4.1299x; 1.2489x over previous
import functools

import jax
import jax.numpy as jnp
from jax import lax
from jax.experimental import pallas as pl
from jax.experimental.pallas import tpu as pltpu
from jax.experimental.pallas import tpu_sc as plsc

N = 320000
D = 128
S = 10000
S_PAD = 10240
EPS = 1e-5

NC = 2
NS = 16
NW = NC * NS
ROWS_W = N // NW
SUB = 80
NSUB = 5
CHUNK = SUB * NSUB
NCHUNK = ROWS_W // CHUNK

R = 2560
NB = N // R
W = 128


def _sc_gather(h_hbm, ids_hbm, g_hbm, idx0, idx1, rows0, rows1,
               gsem0, gsem1, wsem0, wsem1):
    c = lax.axis_index("c")
    s = lax.axis_index("s")
    wid = c * NS + s
    idx_bufs = (idx0, idx1)
    rows_bufs = (rows0, rows1)
    gsems = (gsem0, gsem1)
    wsems = (wsem0, wsem1)

    def fire(k, b):
        pltpu.sync_copy(ids_hbm.at[wid, k], idx_bufs[b])
        for j in range(NSUB):
            pltpu.async_copy(h_hbm.at[idx_bufs[b].at[j]],
                             rows_bufs[b].at[pl.ds(j * SUB, SUB), :], gsems[b])

    def drain_gathers(b):
        for j in range(NSUB):
            pltpu.make_async_copy(
                h_hbm.at[idx_bufs[b].at[j]],
                rows_bufs[b].at[pl.ds(j * SUB, SUB), :], gsems[b]).wait()

    def write(k, b):
        row0 = wid * ROWS_W + k * CHUNK
        pltpu.async_copy(rows_bufs[b], g_hbm.at[pl.ds(row0, CHUNK), :],
                         wsems[b])

    def drain_write(b):
        pltpu.make_async_copy(rows_bufs[b],
                              g_hbm.at[pl.ds(wid * ROWS_W, CHUNK), :],
                              wsems[b]).wait()

    fire(0, 0)

    def body(k, carry):
        for b in range(2):
            @pl.when(k % 2 == b)
            def _step():
                nxt = 1 - b

                @pl.when(k + 1 < NCHUNK)
                def _prefetch():
                    @pl.when(k >= 1)
                    def _dw():
                        drain_write(nxt)

                    fire(k + 1, nxt)

                drain_gathers(b)
                write(k, b)
        return carry

    lax.fori_loop(0, NCHUNK, body, 0)
    drain_write((NCHUNK - 1) % 2)
    drain_write(NCHUNK % 2)


def _sc_gather_call(h, ids4d):
    fn = functools.partial(
        pl.kernel,
        out_type=jax.ShapeDtypeStruct((N, D), jnp.float32),
        mesh=plsc.VectorSubcoreMesh(core_axis_name="c", subcore_axis_name="s",
                                    num_cores=NC, num_subcores=NS),
        scratch_types=[
            pltpu.VMEM((NSUB, SUB), jnp.int32),
            pltpu.VMEM((NSUB, SUB), jnp.int32),
            pltpu.VMEM((CHUNK, D), jnp.float32),
            pltpu.VMEM((CHUNK, D), jnp.float32),
            pltpu.SemaphoreType.DMA,
            pltpu.SemaphoreType.DMA,
            pltpu.SemaphoreType.DMA,
            pltpu.SemaphoreType.DMA,
        ],
    )(_sc_gather)
    return fn(h, ids4d)


def _seg_mlp_kernel(ids_smem_ref, idsr_ref, x_ref, w1_ref, b1_ref, w2_ref,
                    b2_ref, h_ref, acc_ref, cacc_ref):
    i = pl.program_id(0)

    @pl.when(i == 0)
    def _init():
        acc_ref[...] = jnp.zeros((S_PAD, D), jnp.float32)
        cacc_ref[...] = jnp.zeros((S_PAD, D), jnp.float32)

    @pl.when(i < NB)
    def _accum():
        base = ids_smem_ref[0, 0, 0]
        last = ids_smem_ref[0, 0, R - 1]
        off0 = (base // 8) * 8
        nwin = (last - off0) // W + 1

        idsr = idsr_ref[0]
        iota = lax.broadcasted_iota(jnp.int32, (W, R), 0)
        xbf = x_ref[...].astype(jnp.bfloat16)
        ones = jnp.ones((R, D), jnp.bfloat16)
        dims = (((1,), (0,)), ((), ()))

        def accum_win(off):
            oht = (iota == idsr - off).astype(jnp.bfloat16)
            st = jax.lax.dot_general(oht, xbf, dims,
                                     preferred_element_type=jnp.float32)
            ct = jax.lax.dot_general(oht, ones, dims,
                                     preferred_element_type=jnp.float32)
            acc_ref[pl.ds(off, W), :] += st
            cacc_ref[pl.ds(off, W), :] += ct

        accum_win(off0)

        def win_body(k, carry):
            accum_win(off0 + k * W)
            return carry

        @pl.when(nwin > 1)
        def _extra():
            lax.fori_loop(1, nwin, win_body, 0)

    @pl.when(i == NB)
    def _mlp():
        mean = acc_ref[...] / jnp.maximum(cacc_ref[...], 1.0)
        h = jnp.dot(mean, w1_ref[...], preferred_element_type=jnp.float32)
        h = jnp.maximum(h + b1_ref[...], 0.0)
        h = jnp.dot(h, w2_ref[...], preferred_element_type=jnp.float32)
        h_ref[...] = h + b2_ref[...]


def _pass2_kernel(x_ref, g_ref, gamma_ref, beta_ref, out_ref):
    o = x_ref[...] + g_ref[...]
    dims = (((1,), (0,)), ((), ()))
    jd = jnp.full((D, D), 1.0 / D, dtype=jnp.bfloat16)
    mu = jax.lax.dot_general(o.astype(jnp.bfloat16), jd, dims,
                             preferred_element_type=jnp.float32)
    d = o - mu
    msq = jax.lax.dot_general((d * d).astype(jnp.bfloat16), jd, dims,
                              preferred_element_type=jnp.float32)
    rstd = lax.rsqrt(msq + EPS)
    out_ref[...] = d * rstd * gamma_ref[...] + beta_ref[...]


def kernel(x, subnet_id, W1, b1, W2, b2, gamma, beta):
    ids = subnet_id.astype(jnp.int32)
    ids4d = ids.reshape(NW, NCHUNK, NSUB, SUB)
    ids_blk = ids.reshape(NB, 1, R)

    clamp = lambda i: (jnp.minimum(i, NB - 1), 0, 0)
    h = pl.pallas_call(
        _seg_mlp_kernel,
        grid=(NB + 1,),
        in_specs=[
            pl.BlockSpec((1, 1, R), clamp, memory_space=pltpu.SMEM),
            pl.BlockSpec((1, 1, R), clamp),
            pl.BlockSpec((R, D), lambda i: (jnp.minimum(i, NB - 1), 0)),
            pl.BlockSpec((D, D), lambda i: (0, 0)),
            pl.BlockSpec((1, D), lambda i: (0, 0)),
            pl.BlockSpec((D, D), lambda i: (0, 0)),
            pl.BlockSpec((1, D), lambda i: (0, 0)),
        ],
        out_specs=pl.BlockSpec((S_PAD, D), lambda i: (0, 0)),
        out_shape=jax.ShapeDtypeStruct((S_PAD, D), jnp.float32),
        scratch_shapes=[
            pltpu.VMEM((S_PAD, D), jnp.float32),
            pltpu.VMEM((S_PAD, D), jnp.float32),
        ],
    )(ids_blk, ids_blk, x, W1, b1.reshape(1, D), W2, b2.reshape(1, D))

    g = _sc_gather_call(h, ids4d)

    out = pl.pallas_call(
        _pass2_kernel,
        grid=(NB,),
        in_specs=[
            pl.BlockSpec((R, D), lambda i: (i, 0)),
            pl.BlockSpec((R, D), lambda i: (i, 0)),
            pl.BlockSpec((1, D), lambda i: (0, 0)),
            pl.BlockSpec((1, D), lambda i: (0, 0)),
        ],
        out_specs=pl.BlockSpec((R, D), lambda i: (i, 0)),
        out_shape=jax.ShapeDtypeStruct((N, D), jnp.float32),
    )(x, g, gamma.reshape(1, D), beta.reshape(1, D))
    return out


# speedup vs baseline: 4.3611x; 1.0560x over previous
import functools

import jax
import jax.numpy as jnp
from jax import lax
from jax.experimental import pallas as pl
from jax.experimental.pallas import tpu as pltpu
from jax.experimental.pallas import tpu_sc as plsc

N = 320000
D = 128
S = 10000
S_PAD = 10496
EPS = 1e-5

NC = 2
NS = 16
NW = NC * NS
ROWS_W = N // NW
SUB = 80
NSUB = 5
CHUNK = SUB * NSUB
NCHUNK = ROWS_W // CHUNK

R = 4000
NB = N // R
W = 256


def _sc_gather(h_hbm, ids_hbm, g_hbm, idx0, idx1, rows0, rows1,
               gsem0, gsem1, wsem0, wsem1):
    c = lax.axis_index("c")
    s = lax.axis_index("s")
    wid = c * NS + s
    idx_bufs = (idx0, idx1)
    rows_bufs = (rows0, rows1)
    gsems = (gsem0, gsem1)
    wsems = (wsem0, wsem1)

    def fire(k, b):
        pltpu.sync_copy(ids_hbm.at[wid, k], idx_bufs[b])
        for j in range(NSUB):
            pltpu.async_copy(h_hbm.at[idx_bufs[b].at[j]],
                             rows_bufs[b].at[pl.ds(j * SUB, SUB), :], gsems[b])

    def drain_gathers(b):
        for j in range(NSUB):
            pltpu.make_async_copy(
                h_hbm.at[idx_bufs[b].at[j]],
                rows_bufs[b].at[pl.ds(j * SUB, SUB), :], gsems[b]).wait()

    def write(k, b):
        row0 = wid * ROWS_W + k * CHUNK
        pltpu.async_copy(rows_bufs[b], g_hbm.at[pl.ds(row0, CHUNK), :],
                         wsems[b])

    def drain_write(b):
        pltpu.make_async_copy(rows_bufs[b],
                              g_hbm.at[pl.ds(wid * ROWS_W, CHUNK), :],
                              wsems[b]).wait()

    fire(0, 0)

    def body(k, carry):
        for b in range(2):
            @pl.when(k % 2 == b)
            def _step():
                nxt = 1 - b

                @pl.when(k + 1 < NCHUNK)
                def _prefetch():
                    @pl.when(k >= 1)
                    def _dw():
                        drain_write(nxt)

                    fire(k + 1, nxt)

                drain_gathers(b)
                write(k, b)
        return carry

    lax.fori_loop(0, NCHUNK, body, 0)
    drain_write((NCHUNK - 1) % 2)
    drain_write(NCHUNK % 2)


def _sc_gather_call(h, ids4d):
    fn = functools.partial(
        pl.kernel,
        out_type=jax.ShapeDtypeStruct((N, D), jnp.float32),
        mesh=plsc.VectorSubcoreMesh(core_axis_name="c", subcore_axis_name="s",
                                    num_cores=NC, num_subcores=NS),
        scratch_types=[
            pltpu.VMEM((NSUB, SUB), jnp.int32),
            pltpu.VMEM((NSUB, SUB), jnp.int32),
            pltpu.VMEM((CHUNK, D), jnp.float32),
            pltpu.VMEM((CHUNK, D), jnp.float32),
            pltpu.SemaphoreType.DMA,
            pltpu.SemaphoreType.DMA,
            pltpu.SemaphoreType.DMA,
            pltpu.SemaphoreType.DMA,
        ],
    )(_sc_gather)
    return fn(h, ids4d)


def _seg_mlp_kernel(ids_smem_ref, idsr_ref, x_ref, w1_ref, b1_ref, w2_ref,
                    b2_ref, h_ref, acc_ref, cacc_ref):
    i = pl.program_id(0)

    @pl.when(i == 0)
    def _init():
        acc_ref[...] = jnp.zeros((S_PAD, D), jnp.float32)
        cacc_ref[...] = jnp.zeros((S_PAD, D), jnp.float32)

    @pl.when(i < NB)
    def _accum():
        base = ids_smem_ref[0, 0, 0]
        last = ids_smem_ref[0, 0, R - 1]
        off0 = (base // 8) * 8
        nwin = (last - off0) // W + 1

        idsr = idsr_ref[0]
        iota = lax.broadcasted_iota(jnp.int32, (W, R), 0)
        xbf = x_ref[...].astype(jnp.bfloat16)
        ones = jnp.ones((R, D), jnp.bfloat16)
        dims = (((1,), (0,)), ((), ()))

        def accum_win(off):
            oht = (iota == idsr - off).astype(jnp.bfloat16)
            st = jax.lax.dot_general(oht, xbf, dims,
                                     preferred_element_type=jnp.float32)
            ct = jax.lax.dot_general(oht, ones, dims,
                                     preferred_element_type=jnp.float32)
            acc_ref[pl.ds(off, W), :] += st
            cacc_ref[pl.ds(off, W), :] += ct

        accum_win(off0)

        def win_body(k, carry):
            accum_win(off0 + k * W)
            return carry

        @pl.when(nwin > 1)
        def _extra():
            lax.fori_loop(1, nwin, win_body, 0)

    @pl.when(i == NB)
    def _mlp():
        mean = acc_ref[...] / jnp.maximum(cacc_ref[...], 1.0)
        h = jnp.dot(mean, w1_ref[...], preferred_element_type=jnp.float32)
        h = jnp.maximum(h + b1_ref[...], 0.0)
        h = jnp.dot(h, w2_ref[...], preferred_element_type=jnp.float32)
        h_ref[...] = h + b2_ref[...]


def _pass2_kernel(x_ref, g_ref, gamma_ref, beta_ref, out_ref):
    o = x_ref[...] + g_ref[...]
    dims = (((1,), (0,)), ((), ()))
    jd = jnp.full((D, D), 1.0 / D, dtype=jnp.bfloat16)
    mu = jax.lax.dot_general(o.astype(jnp.bfloat16), jd, dims,
                             preferred_element_type=jnp.float32)
    d = o - mu
    msq = jax.lax.dot_general((d * d).astype(jnp.bfloat16), jd, dims,
                              preferred_element_type=jnp.float32)
    rstd = lax.rsqrt(msq + EPS)
    out_ref[...] = d * rstd * gamma_ref[...] + beta_ref[...]


def kernel(x, subnet_id, W1, b1, W2, b2, gamma, beta):
    ids = subnet_id.astype(jnp.int32)
    ids4d = ids.reshape(NW, NCHUNK, NSUB, SUB)
    ids_blk = ids.reshape(NB, 1, R)

    clamp = lambda i: (jnp.minimum(i, NB - 1), 0, 0)
    h = pl.pallas_call(
        _seg_mlp_kernel,
        grid=(NB + 1,),
        in_specs=[
            pl.BlockSpec((1, 1, R), clamp, memory_space=pltpu.SMEM),
            pl.BlockSpec((1, 1, R), clamp),
            pl.BlockSpec((R, D), lambda i: (jnp.minimum(i, NB - 1), 0)),
            pl.BlockSpec((D, D), lambda i: (0, 0)),
            pl.BlockSpec((1, D), lambda i: (0, 0)),
            pl.BlockSpec((D, D), lambda i: (0, 0)),
            pl.BlockSpec((1, D), lambda i: (0, 0)),
        ],
        out_specs=pl.BlockSpec((S_PAD, D), lambda i: (0, 0)),
        out_shape=jax.ShapeDtypeStruct((S_PAD, D), jnp.float32),
        scratch_shapes=[
            pltpu.VMEM((S_PAD, D), jnp.float32),
            pltpu.VMEM((S_PAD, D), jnp.float32),
        ],
    )(ids_blk, ids_blk, x, W1, b1.reshape(1, D), W2, b2.reshape(1, D))

    g = _sc_gather_call(h, ids4d)

    out = pl.pallas_call(
        _pass2_kernel,
        grid=(NB,),
        in_specs=[
            pl.BlockSpec((R, D), lambda i: (i, 0)),
            pl.BlockSpec((R, D), lambda i: (i, 0)),
            pl.BlockSpec((1, D), lambda i: (0, 0)),
            pl.BlockSpec((1, D), lambda i: (0, 0)),
        ],
        out_specs=pl.BlockSpec((R, D), lambda i: (i, 0)),
        out_shape=jax.ShapeDtypeStruct((N, D), jnp.float32),
    )(x, g, gamma.reshape(1, D), beta.reshape(1, D))
    return out
